# baseline (speedup 1.0000x reference)
import functools

import jax
import jax.numpy as jnp
from jax import lax
from jax.experimental import pallas as pl
from jax.experimental.pallas import tpu as pltpu

D_MODEL = 4096
NUM_EXPERTS = 64
TOP_K = 8
CAPACITY_FACTOR = 1.25
ALPHA = 0.01

BLK = 1024


def _gate_kernel(x_ref, wt_ref, b_ref, idx_ref, w_ref, aux_ref, load_acc,
                 *, n_steps, n_tokens):
    i = pl.program_id(0)

    lt = lax.dot_general(wt_ref[...], x_ref[...],
                         (((1,), (1,)), ((), ())),
                         preferred_element_type=jnp.float32)
    lt = lt + b_ref[...]
    rowi = lax.broadcasted_iota(jnp.int32, (NUM_EXPERTS, BLK), 0)
    neg = jnp.float32(-1e30)

    idx_rows = []
    val_rows = []
    pm = lt
    for _ in range(TOP_K):
        mv = jnp.max(pm, axis=0, keepdims=True)
        is_max = pm == mv
        sel = jnp.min(jnp.where(is_max, rowi, NUM_EXPERTS), axis=0,
                      keepdims=True)
        idx_rows.append(sel)
        val_rows.append(mv)
        pm = jnp.where(rowi == sel, neg, pm)

    idx_t = jnp.concatenate(idx_rows, axis=0)
    val_t = jnp.concatenate(val_rows, axis=0)
    e = jnp.exp(val_t - val_t[0:1])
    w_t = e / jnp.sum(e, axis=0, keepdims=True)
    idx_ref[...] = idx_t.T
    w_ref[...] = w_t.T

    sel_mask = (pm <= neg).astype(jnp.float32)
    load_part = jnp.sum(sel_mask, axis=1, keepdims=True)

    @pl.when(i == 0)
    def _init():
        load_acc[...] = jnp.zeros_like(load_acc)

    load_acc[...] += load_part

    @pl.when(i == n_steps - 1)
    def _finalize():
        load = load_acc[...]
        capacity = CAPACITY_FACTOR * (n_tokens * TOP_K) / NUM_EXPERTS
        penalty = jnp.sum(jnp.maximum(load - capacity, 0.0))
        aux = ALPHA * penalty / NUM_EXPERTS / n_tokens
        aux_ref[...] = aux.reshape(1, 1)


def kernel(x, W, b):
    batch, seq, d_model = x.shape
    n_tokens = batch * seq
    xf = x.reshape(n_tokens, d_model)
    wt = W
    n_steps = n_tokens // BLK

    idx, w, aux = pl.pallas_call(
        functools.partial(_gate_kernel, n_steps=n_steps, n_tokens=n_tokens),
        grid=(n_steps,),
        in_specs=[
            pl.BlockSpec((BLK, d_model), lambda i: (i, 0)),
            pl.BlockSpec((NUM_EXPERTS, d_model), lambda i: (0, 0)),
            pl.BlockSpec((NUM_EXPERTS, 1), lambda i: (0, 0)),
        ],
        out_specs=[
            pl.BlockSpec((BLK, TOP_K), lambda i: (i, 0)),
            pl.BlockSpec((BLK, TOP_K), lambda i: (i, 0)),
            pl.BlockSpec((1, 1), lambda i: (0, 0)),
        ],
        out_shape=[
            jax.ShapeDtypeStruct((n_tokens, TOP_K), jnp.int32),
            jax.ShapeDtypeStruct((n_tokens, TOP_K), jnp.float32),
            jax.ShapeDtypeStruct((1, 1), jnp.float32),
        ],
        scratch_shapes=[pltpu.VMEM((NUM_EXPERTS, 1), jnp.float32)],
    )(xf, wt, b.reshape(NUM_EXPERTS, 1))

    return (idx.reshape(batch, seq, TOP_K),
            w.reshape(batch, seq, TOP_K),
            aux[0, 0])


# speedup vs baseline: 1.0127x; 1.0127x over previous
import functools

import jax
import jax.numpy as jnp
from jax import lax
from jax.experimental import pallas as pl
from jax.experimental.pallas import tpu as pltpu

D_MODEL = 4096
NUM_EXPERTS = 64
TOP_K = 8
CAPACITY_FACTOR = 1.25
ALPHA = 0.01

BLK = 1024


def _gate_kernel(x_ref, wt_ref, b_ref, idx_ref, w_ref, aux_ref, load_acc,
                 *, n_steps, n_tokens):
    i = pl.program_id(0)

    logits = lax.dot_general(x_ref[...], wt_ref[...],
                             (((1,), (1,)), ((), ())),
                             preferred_element_type=jnp.float32) + b_ref[...]

    lt = logits.T
    rowi = lax.broadcasted_iota(jnp.int32, (NUM_EXPERTS, BLK), 0)
    neg = jnp.float32(-1e30)

    idx_rows = []
    val_rows = []
    pm = lt
    for _ in range(TOP_K):
        mv = jnp.max(pm, axis=0, keepdims=True)
        is_max = pm == mv
        sel = jnp.min(jnp.where(is_max, rowi, NUM_EXPERTS), axis=0,
                      keepdims=True)
        idx_rows.append(sel)
        val_rows.append(mv)
        pm = jnp.where(rowi == sel, neg, pm)

    idx_t = jnp.concatenate(idx_rows, axis=0)
    val_t = jnp.concatenate(val_rows, axis=0)
    e = jnp.exp(val_t - val_t[0:1])
    w_t = e / jnp.sum(e, axis=0, keepdims=True)
    idx_ref[...] = idx_t.T
    w_ref[...] = w_t.T

    sel_mask = (pm <= neg).astype(jnp.float32)
    load_part = jnp.sum(sel_mask, axis=1, keepdims=True)

    @pl.when(i == 0)
    def _init():
        load_acc[...] = jnp.zeros_like(load_acc)

    load_acc[...] += load_part

    @pl.when(i == n_steps - 1)
    def _finalize():
        load = load_acc[...]
        capacity = CAPACITY_FACTOR * (n_tokens * TOP_K) / NUM_EXPERTS
        penalty = jnp.sum(jnp.maximum(load - capacity, 0.0))
        aux = ALPHA * penalty / NUM_EXPERTS / n_tokens
        aux_ref[...] = aux.reshape(1, 1)


def kernel(x, W, b):
    batch, seq, d_model = x.shape
    n_tokens = batch * seq
    xf = x.reshape(n_tokens, d_model)
    wt = W
    n_steps = n_tokens // BLK

    idx, w, aux = pl.pallas_call(
        functools.partial(_gate_kernel, n_steps=n_steps, n_tokens=n_tokens),
        grid=(n_steps,),
        in_specs=[
            pl.BlockSpec((BLK, d_model), lambda i: (i, 0)),
            pl.BlockSpec((NUM_EXPERTS, d_model), lambda i: (0, 0)),
            pl.BlockSpec((NUM_EXPERTS,), lambda i: (0,)),
        ],
        out_specs=[
            pl.BlockSpec((BLK, TOP_K), lambda i: (i, 0)),
            pl.BlockSpec((BLK, TOP_K), lambda i: (i, 0)),
            pl.BlockSpec((1, 1), lambda i: (0, 0)),
        ],
        out_shape=[
            jax.ShapeDtypeStruct((n_tokens, TOP_K), jnp.int32),
            jax.ShapeDtypeStruct((n_tokens, TOP_K), jnp.float32),
            jax.ShapeDtypeStruct((1, 1), jnp.float32),
        ],
        scratch_shapes=[pltpu.VMEM((NUM_EXPERTS, 1), jnp.float32)],
    )(xf, wt, b)

    return (idx.reshape(batch, seq, TOP_K),
            w.reshape(batch, seq, TOP_K),
            aux[0, 0])
